# baseline (speedup 1.0000x reference)
import jax
import jax.numpy as jnp
from jax.experimental import pallas as pl


def _router_block(x_ref, w_ref, b_ref, logits_ref, probs_ref):
    logits = jnp.dot(x_ref[...], w_ref[...], preferred_element_type=jnp.float32)
    logits = logits + b_ref[...]
    logits_ref[...] = logits
    m = jnp.max(logits, axis=-1, keepdims=True)
    e = jnp.exp(logits - m)
    probs_ref[...] = e / jnp.sum(e, axis=-1, keepdims=True)


def kernel(x, W, b):
    tokens, d = x.shape
    n_adapters = W.shape[1]
    bt = 1024
    b2 = b.reshape(1, n_adapters)
    out_shape = jax.ShapeDtypeStruct((tokens, n_adapters), jnp.float32)
    logits, probs = pl.pallas_call(
        _router_block,
        grid=(tokens // bt,),
        in_specs=[
            pl.BlockSpec((bt, d), lambda i: (i, 0)),
            pl.BlockSpec((d, n_adapters), lambda i: (0, 0)),
            pl.BlockSpec((1, n_adapters), lambda i: (0, 0)),
        ],
        out_specs=[
            pl.BlockSpec((bt, n_adapters), lambda i: (i, 0)),
            pl.BlockSpec((bt, n_adapters), lambda i: (i, 0)),
        ],
        out_shape=[out_shape, out_shape],
    )(x, W, b2)
    return (logits, probs)


# speedup vs baseline: 1.1521x; 1.1521x over previous
import jax
import jax.numpy as jnp
from jax.experimental import pallas as pl


def _router_block(x_ref, w_ref, b_ref, logits_ref, probs_ref):
    logits = jnp.dot(x_ref[...], w_ref[...], preferred_element_type=jnp.float32)
    logits = logits + b_ref[...]
    logits_ref[...] = logits
    m = jnp.max(logits, axis=-1, keepdims=True)
    e = jnp.exp(logits - m)
    probs_ref[...] = e / jnp.sum(e, axis=-1, keepdims=True)


def kernel(x, W, b):
    tokens, d = x.shape
    n_adapters = W.shape[1]
    bt = 4096
    b2 = b.reshape(1, n_adapters)
    out_shape = jax.ShapeDtypeStruct((tokens, n_adapters), jnp.float32)
    logits, probs = pl.pallas_call(
        _router_block,
        grid=(tokens // bt,),
        in_specs=[
            pl.BlockSpec((bt, d), lambda i: (i, 0)),
            pl.BlockSpec((d, n_adapters), lambda i: (0, 0)),
            pl.BlockSpec((1, n_adapters), lambda i: (0, 0)),
        ],
        out_specs=[
            pl.BlockSpec((bt, n_adapters), lambda i: (i, 0)),
            pl.BlockSpec((bt, n_adapters), lambda i: (i, 0)),
        ],
        out_shape=[out_shape, out_shape],
    )(x, W, b2)
    return (logits, probs)
